# baseline (speedup 1.0000x reference)
import functools

import jax
import jax.numpy as jnp
from jax import lax
from jax.experimental import pallas as pl
from jax.experimental.pallas import tpu as pltpu
from jax.experimental.pallas import tpu_sc as plsc

N = 10000
E = 320000
D = 128

NC = 2
NS = 16
NW = NC * NS
CH = 100
NCH = 101
EPW = NCH * CH
EPAD = NW * EPW
RPS = 624
TAIL = N - NS * RPS
LANES = 16


def _sc_spmm_kernel(x_hbm, src_hbm, dst_hbm, ew_hbm, out_hbm,
                    srcb, dstb, ewb, rows0, rows1, hi_sh,
                    gsem0, gsem1, isem0, isem1, isem2, isem3):
    cid = lax.axis_index("c")
    sid = lax.axis_index("s")
    wid = cid * NS + sid
    gsem = (gsem0, gsem1)
    isem = (isem0, isem1, isem2, isem3)
    rowsb = (rows0, rows1)

    def zero_body(r, _):
        for c in range(D // LANES):
            rows0[r, pl.ds(c * LANES, LANES)] = jnp.zeros((LANES,), jnp.float32)
        return 0
    lax.fori_loop(0, CH, zero_body, 0)
    for j in range(RPS // CH):
        pltpu.sync_copy(rows0, hi_sh.at[pl.ds(sid * RPS + j * CH, CH), :])
    rem = RPS - (RPS // CH) * CH
    if rem:
        pltpu.sync_copy(rows0.at[pl.ds(0, rem), :],
                        hi_sh.at[pl.ds(sid * RPS + (RPS // CH) * CH, rem), :])

    @pl.when(sid == NS - 1)
    def _():
        pltpu.sync_copy(rows0.at[pl.ds(0, TAIL), :],
                        hi_sh.at[pl.ds(NS * RPS, TAIL), :])
    plsc.subcore_barrier()

    def fetch_idx(ch, k):
        @pl.when(ch < NCH)
        def _():
            flat = wid * NCH + ch
            pltpu.async_copy(src_hbm.at[flat], srcb.at[k], isem[k])
            pltpu.async_copy(dst_hbm.at[flat], dstb.at[k], isem[k])
            pltpu.async_copy(ew_hbm.at[flat], ewb.at[k], isem[k])

    def wait_idx(ch, k):
        flat = wid * NCH + ch
        pltpu.make_async_copy(src_hbm.at[flat], srcb.at[k], isem[k]).wait()
        pltpu.make_async_copy(dst_hbm.at[flat], dstb.at[k], isem[k]).wait()
        pltpu.make_async_copy(ew_hbm.at[flat], ewb.at[k], isem[k]).wait()

    def start_gather(k, b):
        pltpu.async_copy(x_hbm.at[srcb.at[k]], rowsb[b], gsem[b])

    def process(k, b):
        rows = rowsb[b]
        pltpu.make_async_copy(x_hbm.at[srcb.at[k]], rows, gsem[b]).wait()

        def scale_group(g, _):
            wv = ewb[k, pl.ds(g * LANES, LANES)]
            for j in range(LANES):
                splat = lax.gather(
                    wv, jnp.full((LANES, 1), j, jnp.int32),
                    lax.GatherDimensionNumbers(offset_dims=(),
                                               collapsed_slice_dims=(0,),
                                               start_index_map=(0,)),
                    (1,), mode=lax.GatherScatterMode.PROMISE_IN_BOUNDS)
                e = g * LANES + j
                for c in range(D // LANES):
                    sl = pl.ds(c * LANES, LANES)
                    rows[e, sl] = rows[e, sl] * splat
            return 0
        lax.fori_loop(0, CH // LANES, scale_group, 0)

        pltpu.sync_copy(rows, hi_sh.at[dstb.at[k]], add=True)

    fetch_idx(0, 0)
    fetch_idx(1, 1)
    wait_idx(0, 0)
    start_gather(0, 0)
    fetch_idx(2, 2)

    def quad_body(q, _):
        ch = 4 * q
        wait_idx(ch + 1, 1)
        start_gather(1, 1)
        process(0, 0)
        fetch_idx(ch + 3, 3)
        wait_idx(ch + 2, 2)
        start_gather(2, 0)
        process(1, 1)
        fetch_idx(ch + 4, 0)
        wait_idx(ch + 3, 3)
        start_gather(3, 1)
        process(2, 0)
        fetch_idx(ch + 5, 1)
        wait_idx(ch + 4, 0)
        start_gather(0, 0)
        process(3, 1)
        fetch_idx(ch + 6, 2)
        return 0
    lax.fori_loop(0, (NCH - 1) // 4, quad_body, 0)
    process(0, 0)
    plsc.subcore_barrier()

    pltpu.sync_copy(hi_sh.at[pl.ds(sid * RPS, RPS), :],
                    out_hbm.at[cid, pl.ds(sid * RPS, RPS), :])

    @pl.when(sid == NS - 1)
    def _():
        pltpu.sync_copy(hi_sh.at[pl.ds(NS * RPS, TAIL), :],
                        out_hbm.at[cid, pl.ds(NS * RPS, TAIL), :])


_sc_spmm = functools.partial(
    pl.kernel,
    out_type=jax.ShapeDtypeStruct((NC, N, D), jnp.float32),
    mesh=plsc.VectorSubcoreMesh(core_axis_name="c", subcore_axis_name="s",
                                num_cores=NC, num_subcores=NS),
    scratch_types=[
        pltpu.VMEM((4, CH), jnp.int32),
        pltpu.VMEM((4, CH), jnp.int32),
        pltpu.VMEM((4, CH), jnp.float32),
        pltpu.VMEM((CH, D), jnp.float32),
        pltpu.VMEM((CH, D), jnp.float32),
        pltpu.VMEM_SHARED((N, D), jnp.float32),
    ] + [pltpu.SemaphoreType.DMA] * 6,
)(_sc_spmm_kernel)


BR = 1000


def _tc_body(scal_ref, hi_ref, h0_ref, w_ref, out_ref):
    theta = scal_ref[0]
    alpha = scal_ref[1]
    hi = hi_ref[0] + hi_ref[1]
    support = (1.0 - alpha) * hi + alpha * h0_ref[...]
    out_ref[...] = (theta * jnp.dot(support, w_ref[...],
                                    preferred_element_type=jnp.float32)
                    + (1.0 - theta) * support)


_tc_combine = pl.pallas_call(
    _tc_body,
    grid=(N // BR,),
    in_specs=[
        pl.BlockSpec(memory_space=pltpu.SMEM),
        pl.BlockSpec((NC, BR, D), lambda i: (0, i, 0)),
        pl.BlockSpec((BR, D), lambda i: (i, 0)),
        pl.BlockSpec((D, D), lambda i: (0, 0)),
    ],
    out_specs=pl.BlockSpec((BR, D), lambda i: (i, 0)),
    out_shape=jax.ShapeDtypeStruct((N, D), jnp.float32),
)


def kernel(input, edge_index, edge_weight, h0, W, lamda, alpha, l):
    pad = EPAD - E
    src = jnp.concatenate(
        [edge_index[0].astype(jnp.int32), jnp.zeros((pad,), jnp.int32)]
    ).reshape(NW * NCH, CH)
    dst = jnp.concatenate(
        [edge_index[1].astype(jnp.int32), jnp.zeros((pad,), jnp.int32)]
    ).reshape(NW * NCH, CH)
    ew = jnp.concatenate(
        [edge_weight.astype(jnp.float32), jnp.zeros((pad,), jnp.float32)]
    ).reshape(NW * NCH, CH)
    x = input.astype(jnp.float32)

    hi2 = _sc_spmm(x, src, dst, ew)

    theta = jnp.log(lamda / l + 1.0).astype(jnp.float32)
    alpha_f = jnp.asarray(alpha, jnp.float32)
    scal = jnp.stack([theta, alpha_f])
    return _tc_combine(scal, hi2, h0.astype(jnp.float32), W.astype(jnp.float32))


# speedup vs baseline: 1.1083x; 1.1083x over previous
import functools

import jax
import jax.numpy as jnp
from jax import lax
from jax.experimental import pallas as pl
from jax.experimental.pallas import tpu as pltpu
from jax.experimental.pallas import tpu_sc as plsc

N = 10000
E = 320000
D = 128

NC = 2
NS = 16
NW = NC * NS
CH = 96
NCH = 105
EPW = NCH * CH
EPAD = NW * EPW
RPS = 624
TAIL = N - NS * RPS
LANES = 16


def _sc_spmm_kernel(x_hbm, src_hbm, dst_hbm, ew_hbm, out_hbm,
                    srcb, dstb, ewb, rows0, rows1, hi_sh,
                    gsem0, gsem1, isem0, isem1, isem2, isem3):
    cid = lax.axis_index("c")
    sid = lax.axis_index("s")
    wid = cid * NS + sid
    gsem = (gsem0, gsem1)
    isem = (isem0, isem1, isem2, isem3)
    rowsb = (rows0, rows1)

    def zero_body(r, _):
        for c in range(D // LANES):
            rows0[r, pl.ds(c * LANES, LANES)] = jnp.zeros((LANES,), jnp.float32)
        return 0
    lax.fori_loop(0, CH, zero_body, 0)
    for j in range(RPS // CH):
        pltpu.sync_copy(rows0, hi_sh.at[pl.ds(sid * RPS + j * CH, CH), :])
    rem = RPS - (RPS // CH) * CH
    if rem:
        pltpu.sync_copy(rows0.at[pl.ds(0, rem), :],
                        hi_sh.at[pl.ds(sid * RPS + (RPS // CH) * CH, rem), :])

    @pl.when(sid == NS - 1)
    def _():
        pltpu.sync_copy(rows0.at[pl.ds(0, TAIL), :],
                        hi_sh.at[pl.ds(NS * RPS, TAIL), :])
    plsc.subcore_barrier()

    def fetch_idx(ch, k):
        @pl.when(ch < NCH)
        def _():
            flat = wid * NCH + ch
            pltpu.async_copy(src_hbm.at[flat], srcb.at[k], isem[k])
            pltpu.async_copy(dst_hbm.at[flat], dstb.at[k], isem[k])
            pltpu.async_copy(ew_hbm.at[flat], ewb.at[k], isem[k])

    def wait_idx(ch, k):
        flat = wid * NCH + ch
        pltpu.make_async_copy(src_hbm.at[flat], srcb.at[k], isem[k]).wait()
        pltpu.make_async_copy(dst_hbm.at[flat], dstb.at[k], isem[k]).wait()
        pltpu.make_async_copy(ew_hbm.at[flat], ewb.at[k], isem[k]).wait()

    def start_gather(k, b):
        pltpu.async_copy(x_hbm.at[srcb.at[k]], rowsb[b], gsem[b])

    def process(k, b):
        rows = rowsb[b]
        pltpu.make_async_copy(x_hbm.at[srcb.at[k]], rows, gsem[b]).wait()

        def scale_group(g, _):
            wv = ewb[k, pl.ds(g * LANES, LANES)]
            for j in range(LANES):
                splat = lax.gather(
                    wv, jnp.full((LANES, 1), j, jnp.int32),
                    lax.GatherDimensionNumbers(offset_dims=(),
                                               collapsed_slice_dims=(0,),
                                               start_index_map=(0,)),
                    (1,), mode=lax.GatherScatterMode.PROMISE_IN_BOUNDS)
                e = g * LANES + j
                for c in range(D // LANES):
                    sl = pl.ds(c * LANES, LANES)
                    rows[e, sl] = rows[e, sl] * splat
            return 0
        lax.fori_loop(0, CH // LANES, scale_group, 0)

        pltpu.sync_copy(rows, hi_sh.at[dstb.at[k]], add=True)

    fetch_idx(0, 0)
    fetch_idx(1, 1)
    wait_idx(0, 0)
    start_gather(0, 0)
    fetch_idx(2, 2)

    def quad_body(q, _):
        ch = 4 * q
        wait_idx(ch + 1, 1)
        start_gather(1, 1)
        process(0, 0)
        fetch_idx(ch + 3, 3)
        wait_idx(ch + 2, 2)
        start_gather(2, 0)
        process(1, 1)
        fetch_idx(ch + 4, 0)
        wait_idx(ch + 3, 3)
        start_gather(3, 1)
        process(2, 0)
        fetch_idx(ch + 5, 1)
        wait_idx(ch + 4, 0)
        start_gather(0, 0)
        process(3, 1)
        fetch_idx(ch + 6, 2)
        return 0
    lax.fori_loop(0, (NCH - 1) // 4, quad_body, 0)
    process(0, 0)
    plsc.subcore_barrier()

    pltpu.sync_copy(hi_sh.at[pl.ds(sid * RPS, RPS), :],
                    out_hbm.at[cid, pl.ds(sid * RPS, RPS), :])

    @pl.when(sid == NS - 1)
    def _():
        pltpu.sync_copy(hi_sh.at[pl.ds(NS * RPS, TAIL), :],
                        out_hbm.at[cid, pl.ds(NS * RPS, TAIL), :])


_sc_spmm = functools.partial(
    pl.kernel,
    out_type=jax.ShapeDtypeStruct((NC, N, D), jnp.float32),
    mesh=plsc.VectorSubcoreMesh(core_axis_name="c", subcore_axis_name="s",
                                num_cores=NC, num_subcores=NS),
    scratch_types=[
        pltpu.VMEM((4, CH), jnp.int32),
        pltpu.VMEM((4, CH), jnp.int32),
        pltpu.VMEM((4, CH), jnp.float32),
        pltpu.VMEM((CH, D), jnp.float32),
        pltpu.VMEM((CH, D), jnp.float32),
        pltpu.VMEM_SHARED((N, D), jnp.float32),
    ] + [pltpu.SemaphoreType.DMA] * 6,
)(_sc_spmm_kernel)


BR = 1000


def _tc_body(scal_ref, hi_ref, h0_ref, w_ref, out_ref):
    theta = scal_ref[0]
    alpha = scal_ref[1]
    hi = hi_ref[0] + hi_ref[1]
    support = (1.0 - alpha) * hi + alpha * h0_ref[...]
    out_ref[...] = (theta * jnp.dot(support, w_ref[...],
                                    preferred_element_type=jnp.float32)
                    + (1.0 - theta) * support)


_tc_combine = pl.pallas_call(
    _tc_body,
    grid=(N // BR,),
    in_specs=[
        pl.BlockSpec(memory_space=pltpu.SMEM),
        pl.BlockSpec((NC, BR, D), lambda i: (0, i, 0)),
        pl.BlockSpec((BR, D), lambda i: (i, 0)),
        pl.BlockSpec((D, D), lambda i: (0, 0)),
    ],
    out_specs=pl.BlockSpec((BR, D), lambda i: (i, 0)),
    out_shape=jax.ShapeDtypeStruct((N, D), jnp.float32),
)


def kernel(input, edge_index, edge_weight, h0, W, lamda, alpha, l):
    pad = EPAD - E
    src = jnp.concatenate(
        [edge_index[0].astype(jnp.int32), jnp.zeros((pad,), jnp.int32)]
    ).reshape(NW * NCH, CH)
    dst = jnp.concatenate(
        [edge_index[1].astype(jnp.int32), jnp.zeros((pad,), jnp.int32)]
    ).reshape(NW * NCH, CH)
    ew = jnp.concatenate(
        [edge_weight.astype(jnp.float32), jnp.zeros((pad,), jnp.float32)]
    ).reshape(NW * NCH, CH)
    x = input.astype(jnp.float32)

    hi2 = _sc_spmm(x, src, dst, ew)

    theta = jnp.log(lamda / l + 1.0).astype(jnp.float32)
    alpha_f = jnp.asarray(alpha, jnp.float32)
    scal = jnp.stack([theta, alpha_f])
    return _tc_combine(scal, hi2, h0.astype(jnp.float32), W.astype(jnp.float32))


# speedup vs baseline: 1.6135x; 1.4559x over previous
import functools

import jax
import jax.numpy as jnp
from jax import lax
from jax.experimental import pallas as pl
from jax.experimental.pallas import tpu as pltpu
from jax.experimental.pallas import tpu_sc as plsc

N = 10000
E = 320000
D = 128

NC = 2
NS = 16
NW = NC * NS
CH = 96
NCH = 105
EPW = NCH * CH
EPAD = NW * EPW
RPS = 624
TAIL = N - NS * RPS
LANES = 16


def _sc_spmm_kernel(x_hbm, src_hbm, dst_hbm, ew_hbm, out_hbm,
                    srcb, dstb, ewb, rows0, rows1, hi_sh,
                    gsem0, gsem1, isem0, isem1, isem2, isem3):
    cid = lax.axis_index("c")
    sid = lax.axis_index("s")
    wid = cid * NS + sid
    gsem = (gsem0, gsem1)
    isem = (isem0, isem1, isem2, isem3)
    rowsb = (rows0, rows1)

    def zero_body(r, _):
        for c in range(D // LANES):
            rows0[r, pl.ds(c * LANES, LANES)] = jnp.zeros((LANES,), jnp.float32)
        return 0
    lax.fori_loop(0, CH, zero_body, 0)
    for j in range(RPS // CH):
        pltpu.sync_copy(rows0, hi_sh.at[pl.ds(sid * RPS + j * CH, CH), :])
    rem = RPS - (RPS // CH) * CH
    if rem:
        pltpu.sync_copy(rows0.at[pl.ds(0, rem), :],
                        hi_sh.at[pl.ds(sid * RPS + (RPS // CH) * CH, rem), :])

    @pl.when(sid == NS - 1)
    def _():
        pltpu.sync_copy(rows0.at[pl.ds(0, TAIL), :],
                        hi_sh.at[pl.ds(NS * RPS, TAIL), :])
    plsc.subcore_barrier()

    def fetch_idx(ch, k):
        @pl.when(ch < NCH)
        def _():
            flat = wid * NCH + ch
            pltpu.async_copy(src_hbm.at[flat], srcb.at[k], isem[k])
            pltpu.async_copy(dst_hbm.at[flat], dstb.at[k], isem[k])
            pltpu.async_copy(ew_hbm.at[flat], ewb.at[k], isem[k])

    def wait_idx(ch, k):
        flat = wid * NCH + ch
        pltpu.make_async_copy(src_hbm.at[flat], srcb.at[k], isem[k]).wait()
        pltpu.make_async_copy(dst_hbm.at[flat], dstb.at[k], isem[k]).wait()
        pltpu.make_async_copy(ew_hbm.at[flat], ewb.at[k], isem[k]).wait()

    def start_gather(k, b):
        pltpu.async_copy(x_hbm.at[srcb.at[k]], rowsb[b], gsem[b])

    def process(k, b):
        rows = rowsb[b]
        pltpu.make_async_copy(x_hbm.at[srcb.at[k]], rows, gsem[b]).wait()

        def scale_group(g, _):
            wv = ewb[k, pl.ds(g * LANES, LANES)]
            for j in range(LANES):
                splat = lax.gather(
                    wv, jnp.full((LANES, 1), j, jnp.int32),
                    lax.GatherDimensionNumbers(offset_dims=(),
                                               collapsed_slice_dims=(0,),
                                               start_index_map=(0,)),
                    (1,), mode=lax.GatherScatterMode.PROMISE_IN_BOUNDS)
                e = g * LANES + j
                for c in range(D // LANES):
                    sl = pl.ds(c * LANES, LANES)
                    rows[e, sl] = rows[e, sl] * splat
            return 0
        lax.fori_loop(0, CH // LANES, scale_group, 0)

        pltpu.sync_copy(rows, hi_sh.at[dstb.at[k]], add=True)

    fetch_idx(0, 0)
    fetch_idx(1, 1)
    wait_idx(0, 0)
    start_gather(0, 0)
    fetch_idx(2, 2)

    def quad_body(q, _):
        ch = 4 * q
        wait_idx(ch + 1, 1)
        start_gather(1, 1)
        process(0, 0)
        fetch_idx(ch + 3, 3)
        wait_idx(ch + 2, 2)
        start_gather(2, 0)
        process(1, 1)
        fetch_idx(ch + 4, 0)
        wait_idx(ch + 3, 3)
        start_gather(3, 1)
        process(2, 0)
        fetch_idx(ch + 5, 1)
        wait_idx(ch + 4, 0)
        start_gather(0, 0)
        process(3, 1)
        fetch_idx(ch + 6, 2)
        return 0
    lax.fori_loop(0, (NCH - 1) // 4, quad_body, 0)
    process(0, 0)
    plsc.subcore_barrier()

    pltpu.sync_copy(hi_sh.at[pl.ds(sid * RPS, RPS), :],
                    out_hbm.at[cid, pl.ds(sid * RPS, RPS), :])

    @pl.when(sid == NS - 1)
    def _():
        pltpu.sync_copy(hi_sh.at[pl.ds(NS * RPS, TAIL), :],
                        out_hbm.at[cid, pl.ds(NS * RPS, TAIL), :])


_sc_spmm = functools.partial(
    pl.kernel,
    out_type=jax.ShapeDtypeStruct((NC, N, D), jnp.float32),
    mesh=plsc.VectorSubcoreMesh(core_axis_name="c", subcore_axis_name="s",
                                num_cores=NC, num_subcores=NS),
    scratch_types=[
        pltpu.VMEM((4, CH), jnp.int32),
        pltpu.VMEM((4, CH), jnp.int32),
        pltpu.VMEM((4, CH), jnp.float32),
        pltpu.VMEM((CH, D), jnp.float32),
        pltpu.VMEM((CH, D), jnp.float32),
        pltpu.VMEM_SHARED((N, D), jnp.float32),
    ] + [pltpu.SemaphoreType.DMA] * 6,
)(_sc_spmm_kernel)


BR = 1000


def _tc_body(scal_ref, hi_ref, h0_ref, w_ref, out_ref):
    theta = scal_ref[0]
    alpha = scal_ref[1]
    hi = hi_ref[0] + hi_ref[1]
    support = (1.0 - alpha) * hi + alpha * h0_ref[...]
    out_ref[...] = (theta * jnp.dot(support, w_ref[...],
                                    preferred_element_type=jnp.float32)
                    + (1.0 - theta) * support)


_tc_combine = pl.pallas_call(
    _tc_body,
    grid=(N // BR,),
    in_specs=[
        pl.BlockSpec(memory_space=pltpu.SMEM),
        pl.BlockSpec((NC, BR, D), lambda i: (0, i, 0)),
        pl.BlockSpec((BR, D), lambda i: (i, 0)),
        pl.BlockSpec((D, D), lambda i: (0, 0)),
    ],
    out_specs=pl.BlockSpec((BR, D), lambda i: (i, 0)),
    out_shape=jax.ShapeDtypeStruct((N, D), jnp.float32),
)


def kernel(input, edge_index, edge_weight, h0, W, lamda, alpha, l):
    pad = EPAD - E
    spread = (jnp.arange(pad, dtype=jnp.int32) * 13) % N
    src = jnp.concatenate(
        [edge_index[0].astype(jnp.int32), spread]
    ).reshape(NW * NCH, CH)
    dst = jnp.concatenate(
        [edge_index[1].astype(jnp.int32), spread]
    ).reshape(NW * NCH, CH)
    ew = jnp.concatenate(
        [edge_weight.astype(jnp.float32), jnp.zeros((pad,), jnp.float32)]
    ).reshape(NW * NCH, CH)
    x = input.astype(jnp.float32)

    hi2 = _sc_spmm(x, src, dst, ew)

    theta = jnp.log(lamda / l + 1.0).astype(jnp.float32)
    alpha_f = jnp.asarray(alpha, jnp.float32)
    scal = jnp.stack([theta, alpha_f])
    return _tc_combine(scal, hi2, h0.astype(jnp.float32), W.astype(jnp.float32))
